# baseline (speedup 1.0000x reference)
import functools

import jax
import jax.numpy as jnp
from jax import lax
from jax.experimental import pallas as pl
from jax.experimental.pallas import tpu as pltpu
from jax.experimental.pallas import tpu_sc as plsc

_LANES = 16
_NUM_WORKERS = 32
_NUM_CORES = 2


def _sc_body(b_per_w, d, x_hbm, idx_hbm, bank_hbm, out_hbm,
             idx_v, nbuf, xbuf, sem_n, sem_x):
    wid = lax.axis_index("s") * _NUM_CORES + lax.axis_index("c")
    base = wid * b_per_w
    pltpu.sync_copy(idx_hbm.at[pl.ds(base, b_per_w)], idx_v)

    def row(j, carry):
        cp_n = pltpu.async_copy(bank_hbm.at[idx_v.at[j]], nbuf, sem_n)
        cp_x = pltpu.async_copy(x_hbm.at[pl.ds(base + j, 1)], xbuf, sem_x)
        cp_n.wait()
        cp_x.wait()

        def add(i, c):
            sl = pl.ds(i * _LANES, _LANES)
            xbuf[0, sl] = xbuf[0, sl] + nbuf[0, sl]
            return c

        lax.fori_loop(0, d // _LANES, add, 0, unroll=8)
        pltpu.sync_copy(xbuf, out_hbm.at[pl.ds(base + j, 1)])
        return carry

    lax.fori_loop(0, b_per_w, row, 0)


@functools.partial(jax.jit, static_argnames=())
def kernel(x, indices, noise_bank):
    B, C, T = x.shape
    D = C * T
    V = noise_bank.shape[0]
    b_per_w = B // _NUM_WORKERS

    x2 = x.reshape(B, D)
    bank2 = noise_bank.reshape(V, D)
    idx = indices.astype(jnp.int32).reshape(B, 1)

    mesh = plsc.VectorSubcoreMesh(core_axis_name="c", subcore_axis_name="s")
    run = pl.kernel(
        functools.partial(_sc_body, b_per_w, D),
        out_type=jax.ShapeDtypeStruct((B, D), jnp.float32),
        scratch_types=[
            pltpu.VMEM((b_per_w, 1), jnp.int32),
            pltpu.VMEM((1, D), jnp.float32),
            pltpu.VMEM((1, D), jnp.float32),
            pltpu.SemaphoreType.DMA,
            pltpu.SemaphoreType.DMA,
        ],
        mesh=mesh,
    )
    out = run(x2, idx, bank2)
    return out.reshape(B, C, T)
